# baseline (speedup 1.0000x reference)
import functools

import jax
import jax.numpy as jnp
from jax import lax
from jax.experimental import pallas as pl
from jax.experimental.pallas import tpu as pltpu

_DELTA_V = 0.5
_DELTA_D = 3.0
_NL = 4
_R = 8


def _pick_sub(hw):
    for c in (8192, 4096, 2048, 1024, 512, 256, 128):
        if hw % c == 0:
            return c
    return hw


def _body(emb_ref, lab_ref, var_ref, dist_ref,
          stats_ref, statsb_ref, accb_ref, acc_ref, *, nb, nsub, sub):
    b = pl.program_id(0)
    phase = pl.program_id(1)
    c = emb_ref.shape[1]

    def lane_masks(i):
        lab = lab_ref[0, :, pl.ds(i * sub, sub)]
        lane_ids = lax.broadcasted_iota(jnp.int32, (_R, sub), 0) + 1
        return (jnp.broadcast_to(lab, (_R, sub)) == lane_ids).astype(
            jnp.float32)

    @pl.when(phase == 0)
    def _():
        statsb_ref[...] = jnp.zeros_like(statsb_ref)
        ones_row = jnp.ones((1, sub), dtype=jnp.float32)

        def step(i, carry):
            emb = emb_ref[0, :, pl.ds(i * sub, sub)]
            masks = lane_masks(i)
            sums = lax.dot_general(
                masks, emb, (((1,), (1,)), ((), ())),
                preferred_element_type=jnp.float32)
            counts = lax.dot_general(
                masks, ones_row, (((1,), (1,)), ((), ())),
                preferred_element_type=jnp.float32)
            pad = jnp.zeros((_R, 128 - c - 1), dtype=jnp.float32)
            statsb_ref[...] += jnp.concatenate([sums, counts, pad], axis=1)
            return carry

        lax.fori_loop(0, nsub, step, 0)
        stats_ref[pl.ds(b * _R, _R), :] = statsb_ref[...]

    @pl.when(phase == 1)
    def _():
        accb_ref[...] = jnp.zeros_like(accb_ref)
        stats_b = stats_ref[pl.ds(b * _R, _R), :]
        cnt = stats_b[:, c:c + 1]
        safe_cnt = jnp.where(cnt > 0, cnt, 1.0)
        mu = stats_b[:, 0:c] / safe_cnt
        sq_mu = jnp.sum(mu * mu, axis=1, keepdims=True)
        neg2mu = -2.0 * mu
        ones_sq = jnp.ones((_R, c), dtype=jnp.float32)

        def step(i, carry):
            emb = emb_ref[0, :, pl.ds(i * sub, sub)]
            masks = lane_masks(i)
            sq_e = lax.dot_general(
                ones_sq, emb * emb, (((1,), (0,)), ((), ())),
                preferred_element_type=jnp.float32)
            dots = lax.dot_general(
                neg2mu, emb, (((1,), (0,)), ((), ())),
                preferred_element_type=jnp.float32)
            d2 = jnp.maximum(sq_e + dots + sq_mu, 0.0)
            d = jnp.sqrt(d2)
            hinge = jnp.maximum(d - _DELTA_V, 0.0)
            accb_ref[...] += hinge * hinge * masks
            return carry

        lax.fori_loop(0, nsub, step, 0)
        lane_sums = jnp.sum(accb_ref[...], axis=1, keepdims=True)
        pad = jnp.zeros((_R, 127), dtype=jnp.float32)
        acc_ref[pl.ds(b * _R, _R), :] = jnp.concatenate(
            [lane_sums, pad], axis=1)

        @pl.when(b == nb - 1)
        def _():
            var_loss = jnp.float32(0.0)
            dist_loss = jnp.float32(0.0)
            for bb in range(nb):
                stats_bb = stats_ref[bb * _R:(bb + 1) * _R, :]
                cnt_b = stats_bb[0:_NL, c:c + 1]
                has = cnt_b > 0
                safe = jnp.where(has, cnt_b, 1.0)
                varsums = acc_ref[bb * _R:bb * _R + _NL, 0:1]
                batch_var = jnp.sum(jnp.where(has, varsums / safe, 0.0))
                nl = jnp.sum(has.astype(jnp.float32))
                mu_b = jnp.where(has, stats_bb[0:_NL, 0:c] / safe, 0.0)
                contrib = jnp.float32(0.0)
                for i in range(_NL):
                    for k in range(i + 1, _NL):
                        diff = mu_b[i:i + 1, :] - mu_b[k:k + 1, :]
                        pd2 = jnp.sum(diff * diff)
                        pd = jnp.where(pd2 > 0,
                                       jnp.sqrt(jnp.where(pd2 > 0, pd2, 1.0)),
                                       0.0)
                        both = (cnt_b[i, 0] * cnt_b[k, 0]) > 0
                        h = jnp.maximum(_DELTA_D - pd, 0.0)
                        contrib += 2.0 * jnp.where(both, h * h, 0.0)
                new_var = (var_loss + batch_var) / nl
                var_loss = jnp.where(nl > 0, new_var, var_loss)
                new_dist = (dist_loss + jnp.where(nl > 1, contrib, 0.0)) / (
                    2.0 * nl * (nl - 1.0))
                dist_loss = jnp.where(nl > 0, new_dist, dist_loss)
            var_ref[...] = jnp.reshape(var_loss / nb, (1, 1))
            dist_ref[...] = jnp.reshape(dist_loss / nb, (1, 1))


def _run(emb3, lab3, interpret=False):
    nb, c, hw = emb3.shape
    sub = _pick_sub(hw)
    nsub = hw // sub
    grid = (nb, 2)

    var, dist = pl.pallas_call(
        functools.partial(_body, nb=nb, nsub=nsub, sub=sub),
        grid=grid,
        in_specs=[pl.BlockSpec((1, c, hw), lambda b, p: (b, 0, 0)),
                  pl.BlockSpec((1, 1, hw), lambda b, p: (b, 0, 0))],
        out_specs=[pl.BlockSpec((1, 1), lambda b, p: (0, 0)),
                   pl.BlockSpec((1, 1), lambda b, p: (0, 0))],
        out_shape=[jax.ShapeDtypeStruct((1, 1), jnp.float32),
                   jax.ShapeDtypeStruct((1, 1), jnp.float32)],
        scratch_shapes=[pltpu.VMEM((_R * nb, 128), jnp.float32),
                        pltpu.VMEM((_R, 128), jnp.float32),
                        pltpu.VMEM((_R, sub), jnp.float32),
                        pltpu.VMEM((_R * nb, 128), jnp.float32)],
        compiler_params=pltpu.CompilerParams(
            dimension_semantics=("arbitrary", "arbitrary")),
        interpret=interpret,
    )(emb3, lab3)

    return var[0, 0], dist[0, 0]


def kernel(embedding_tensor, instance_labels):
    nb, c, h, w = embedding_tensor.shape
    emb3 = embedding_tensor.reshape(nb, c, h * w)
    lab3 = instance_labels.reshape(nb, 1, h * w).astype(jnp.int32)
    return _run(emb3, lab3)


# speedup vs baseline: 1.2866x; 1.2866x over previous
import functools

import jax
import jax.numpy as jnp
from jax import lax
from jax.experimental import pallas as pl
from jax.experimental.pallas import tpu as pltpu

_DELTA_V = 0.5
_DELTA_D = 3.0
_NL = 4
_R = 8


def _pick_sub(hw):
    for c in (8192, 4096, 2048, 1024, 512, 256, 128):
        if hw % c == 0:
            return c
    return hw


def _body(emb_ref, lab_ref, var_ref, dist_ref,
          stats_ref, statsb_ref, accb_ref, acc_ref, *, nb, nsub, sub):
    b = pl.program_id(0)
    phase = pl.program_id(1)
    c = emb_ref.shape[1]

    def lane_masks(i):
        lab = lab_ref[0, :, pl.ds(i * sub, sub)]
        lane_ids = lax.broadcasted_iota(jnp.int32, (_R, sub), 0) + 1
        return (jnp.broadcast_to(lab, (_R, sub)) == lane_ids).astype(
            jnp.float32)

    @pl.when(phase == 0)
    def _():
        statsb_ref[...] = jnp.zeros_like(statsb_ref)
        ones_row = jnp.ones((1, sub), dtype=jnp.float32)

        def step(i, carry):
            emb = emb_ref[0, :, pl.ds(i * sub, sub)]
            masks = lane_masks(i)
            sums = lax.dot_general(
                masks, emb, (((1,), (1,)), ((), ())),
                preferred_element_type=jnp.float32)
            counts = lax.dot_general(
                masks, ones_row, (((1,), (1,)), ((), ())),
                preferred_element_type=jnp.float32)
            pad = jnp.zeros((_R, 128 - c - 1), dtype=jnp.float32)
            statsb_ref[...] += jnp.concatenate([sums, counts, pad], axis=1)
            return carry

        lax.fori_loop(0, nsub, step, 0, unroll=4)
        stats_ref[pl.ds(b * _R, _R), :] = statsb_ref[...]

    @pl.when(phase == 1)
    def _():
        accb_ref[...] = jnp.zeros_like(accb_ref)
        stats_b = stats_ref[pl.ds(b * _R, _R), :]
        cnt = stats_b[:, c:c + 1]
        safe_cnt = jnp.where(cnt > 0, cnt, 1.0)
        mu = stats_b[:, 0:c] / safe_cnt
        sq_mu = jnp.sum(mu * mu, axis=1, keepdims=True)
        neg2mu = -2.0 * mu
        ones_sq = jnp.ones((_R, c), dtype=jnp.float32)

        def step(i, carry):
            emb = emb_ref[0, :, pl.ds(i * sub, sub)]
            masks = lane_masks(i)
            sq_e = lax.dot_general(
                ones_sq, emb * emb, (((1,), (0,)), ((), ())),
                preferred_element_type=jnp.float32)
            dots = lax.dot_general(
                neg2mu, emb, (((1,), (0,)), ((), ())),
                preferred_element_type=jnp.float32)
            d2 = jnp.maximum(sq_e + dots + sq_mu, 0.0)
            d = jnp.sqrt(d2)
            hinge = jnp.maximum(d - _DELTA_V, 0.0)
            accb_ref[...] += hinge * hinge * masks
            return carry

        lax.fori_loop(0, nsub, step, 0, unroll=4)
        lane_sums = jnp.sum(accb_ref[...], axis=1, keepdims=True)
        pad = jnp.zeros((_R, 127), dtype=jnp.float32)
        acc_ref[pl.ds(b * _R, _R), :] = jnp.concatenate(
            [lane_sums, pad], axis=1)

        @pl.when(b == nb - 1)
        def _():
            var_loss = jnp.float32(0.0)
            dist_loss = jnp.float32(0.0)
            for bb in range(nb):
                stats_bb = stats_ref[bb * _R:(bb + 1) * _R, :]
                cnt_b = stats_bb[0:_NL, c:c + 1]
                has = cnt_b > 0
                safe = jnp.where(has, cnt_b, 1.0)
                varsums = acc_ref[bb * _R:bb * _R + _NL, 0:1]
                batch_var = jnp.sum(jnp.where(has, varsums / safe, 0.0))
                nl = jnp.sum(has.astype(jnp.float32))
                mu_b = jnp.where(has, stats_bb[0:_NL, 0:c] / safe, 0.0)
                contrib = jnp.float32(0.0)
                for i in range(_NL):
                    for k in range(i + 1, _NL):
                        diff = mu_b[i:i + 1, :] - mu_b[k:k + 1, :]
                        pd2 = jnp.sum(diff * diff)
                        pd = jnp.where(pd2 > 0,
                                       jnp.sqrt(jnp.where(pd2 > 0, pd2, 1.0)),
                                       0.0)
                        both = (cnt_b[i, 0] * cnt_b[k, 0]) > 0
                        h = jnp.maximum(_DELTA_D - pd, 0.0)
                        contrib += 2.0 * jnp.where(both, h * h, 0.0)
                new_var = (var_loss + batch_var) / nl
                var_loss = jnp.where(nl > 0, new_var, var_loss)
                new_dist = (dist_loss + jnp.where(nl > 1, contrib, 0.0)) / (
                    2.0 * nl * (nl - 1.0))
                dist_loss = jnp.where(nl > 0, new_dist, dist_loss)
            var_ref[...] = jnp.reshape(var_loss / nb, (1, 1))
            dist_ref[...] = jnp.reshape(dist_loss / nb, (1, 1))


def _run(emb3, lab3, interpret=False):
    nb, c, hw = emb3.shape
    sub = _pick_sub(hw)
    nsub = hw // sub
    grid = (nb, 2)

    var, dist = pl.pallas_call(
        functools.partial(_body, nb=nb, nsub=nsub, sub=sub),
        grid=grid,
        in_specs=[pl.BlockSpec((1, c, hw), lambda b, p: (b, 0, 0)),
                  pl.BlockSpec((1, 1, hw), lambda b, p: (b, 0, 0))],
        out_specs=[pl.BlockSpec((1, 1), lambda b, p: (0, 0)),
                   pl.BlockSpec((1, 1), lambda b, p: (0, 0))],
        out_shape=[jax.ShapeDtypeStruct((1, 1), jnp.float32),
                   jax.ShapeDtypeStruct((1, 1), jnp.float32)],
        scratch_shapes=[pltpu.VMEM((_R * nb, 128), jnp.float32),
                        pltpu.VMEM((_R, 128), jnp.float32),
                        pltpu.VMEM((_R, sub), jnp.float32),
                        pltpu.VMEM((_R * nb, 128), jnp.float32)],
        compiler_params=pltpu.CompilerParams(
            dimension_semantics=("arbitrary", "arbitrary")),
        interpret=interpret,
    )(emb3, lab3)

    return var[0, 0], dist[0, 0]


def kernel(embedding_tensor, instance_labels):
    nb, c, h, w = embedding_tensor.shape
    emb3 = embedding_tensor.reshape(nb, c, h * w)
    lab3 = instance_labels.reshape(nb, 1, h * w).astype(jnp.int32)
    return _run(emb3, lab3)
